# Initial kernel scaffold; baseline (speedup 1.0000x reference)
import jax
import jax.numpy as jnp
from jax import lax
from jax.experimental import pallas as pl
from jax.experimental.pallas import tpu as pltpu
from jax.experimental.pallas import tpu_sc as plsc

B, S, D = 32, 512, 768
T = B * S
NC, NS = 2, 16
NW = NC * NS
TOK_PER_W = T // NW
F = 64
NSUB = TOK_PER_W // F


def _sc_gather_body(table_hbm, ids_hbm, out_hbm, idx_v, rows_v, sem):
    wid = lax.axis_index("s") * NC + lax.axis_index("c")
    base = wid * TOK_PER_W
    for f in range(NSUB):
        off = base + f * F
        pltpu.sync_copy(ids_hbm.at[pl.ds(off, F)], idx_v)
        pltpu.async_copy(table_hbm.at[idx_v], rows_v, sem).wait()
        pltpu.sync_copy(rows_v, out_hbm.at[pl.ds(off, F)])


_sc_gather = pl.kernel(
    _sc_gather_body,
    out_type=jax.ShapeDtypeStruct((T, D), jnp.float32),
    mesh=plsc.VectorSubcoreMesh(
        core_axis_name="c", subcore_axis_name="s", num_cores=NC, num_subcores=NS
    ),
    scratch_types=[
        pltpu.VMEM((F,), jnp.int32),
        pltpu.VMEM((F, D), jnp.float32),
        pltpu.SemaphoreType.DMA,
    ],
)

BT = 512


def _tc_ln_body(g_ref, tts_ref, pos_ref, wt_ref, gamma_ref, beta_ref, out_ref):
    x = g_ref[...]
    tts = tts_ref[0, 0, :]
    w0 = wt_ref[0:1, :]
    w1 = wt_ref[1:2, :]
    typ = jnp.where((tts == 0)[:, None], w0, w1)
    x = x + pos_ref[...] + typ
    mean = jnp.mean(x, axis=-1, keepdims=True)
    xc = x - mean
    var = jnp.mean(xc * xc, axis=-1, keepdims=True)
    normed = xc * lax.rsqrt(var + 1e-12)
    out_ref[...] = normed * gamma_ref[...] + beta_ref[...]


def kernel(input_ids, token_type_ids, W_word, W_pos, W_type, gamma, beta):
    ids_flat = input_ids.reshape(T).astype(jnp.int32)
    tts = token_type_ids.reshape(T // BT, 1, BT).astype(jnp.int32)

    gathered = _sc_gather(W_word, ids_flat)

    out = pl.pallas_call(
        _tc_ln_body,
        grid=(T // BT,),
        in_specs=[
            pl.BlockSpec((BT, D), lambda i: (i, 0)),
            pl.BlockSpec((1, 1, BT), lambda i: (i, 0, 0)),
            pl.BlockSpec((BT, D), lambda i: (i % (S // BT), 0)),
            pl.BlockSpec((2, D), lambda i: (0, 0)),
            pl.BlockSpec((1, D), lambda i: (0, 0)),
            pl.BlockSpec((1, D), lambda i: (0, 0)),
        ],
        out_specs=pl.BlockSpec((BT, D), lambda i: (i, 0)),
        out_shape=jax.ShapeDtypeStruct((T, D), jnp.float32),
    )(gathered, tts, W_pos, W_type, gamma.reshape(1, D), beta.reshape(1, D))

    return out.reshape(B, S, D)


# speedup vs baseline: 1.7446x; 1.7446x over previous
import jax
import jax.numpy as jnp
from jax import lax
from jax.experimental import pallas as pl
from jax.experimental.pallas import tpu as pltpu
from jax.experimental.pallas import tpu_sc as plsc

B, S, D = 32, 512, 768
T = B * S
NC, NS = 2, 16
NW = NC * NS
TOK_PER_W = T // NW
F = 64
NSUB = TOK_PER_W // F


def _sc_gather_body(table_hbm, ids_hbm, out_hbm, idx_v, rows_v, sem):
    wid = lax.axis_index("s") * NC + lax.axis_index("c")
    base = wid * TOK_PER_W
    for f in range(NSUB):
        off = base + f * F
        pltpu.sync_copy(ids_hbm.at[pl.ds(off, F)], idx_v)
        pltpu.async_copy(table_hbm.at[idx_v], rows_v, sem).wait()
        pltpu.sync_copy(rows_v, out_hbm.at[pl.ds(off, F)])


_sc_gather = pl.kernel(
    _sc_gather_body,
    out_type=jax.ShapeDtypeStruct((T, D), jnp.float32),
    mesh=plsc.VectorSubcoreMesh(
        core_axis_name="c", subcore_axis_name="s", num_cores=NC, num_subcores=NS
    ),
    scratch_types=[
        pltpu.VMEM((F,), jnp.int32),
        pltpu.VMEM((F, D), jnp.float32),
        pltpu.SemaphoreType.DMA,
    ],
)

BT = 512


def _tc_ln_body(g_ref, tts_ref, pos_ref, wt_ref, gamma_ref, beta_ref, out_ref):
    x = g_ref[...]
    t = tts_ref[...]
    w0 = wt_ref[0:1, :]
    diff = wt_ref[1:2, :] - w0
    x = x + pos_ref[...] + w0 + t * diff
    mean = jnp.mean(x, axis=-1, keepdims=True)
    xc = x - mean
    var = jnp.mean(xc * xc, axis=-1, keepdims=True)
    normed = xc * lax.rsqrt(var + 1e-12)
    out_ref[...] = normed * gamma_ref[...] + beta_ref[...]


def kernel(input_ids, token_type_ids, W_word, W_pos, W_type, gamma, beta):
    ids_flat = input_ids.reshape(T).astype(jnp.int32)
    tts = token_type_ids.reshape(T, 1).astype(jnp.float32)

    gathered = _sc_gather(W_word, ids_flat)

    out = pl.pallas_call(
        _tc_ln_body,
        grid=(T // BT,),
        in_specs=[
            pl.BlockSpec((BT, D), lambda i: (i, 0)),
            pl.BlockSpec((BT, 1), lambda i: (i, 0)),
            pl.BlockSpec((BT, D), lambda i: (i % (S // BT), 0)),
            pl.BlockSpec((2, D), lambda i: (0, 0)),
            pl.BlockSpec((1, D), lambda i: (0, 0)),
            pl.BlockSpec((1, D), lambda i: (0, 0)),
        ],
        out_specs=pl.BlockSpec((BT, D), lambda i: (i, 0)),
        out_shape=jax.ShapeDtypeStruct((T, D), jnp.float32),
    )(gathered, tts, W_pos, W_type, gamma.reshape(1, D), beta.reshape(1, D))

    return out.reshape(B, S, D)
